# baseline (speedup 1.0000x reference)
import functools

import jax
import jax.numpy as jnp
from jax import lax
from jax.experimental import pallas as pl
from jax.experimental.pallas import tpu as pltpu
from jax.experimental.pallas import tpu_sc as plsc

N_NODES = 10000
N_EDGES = 320000
D_FEAT = 128
SCALE = float(32.0 ** -0.5)

NC = 2
NS = 16
NW = NC * NS
CH = 128
NCHUNKS = N_EDGES // CH
CPT = NCHUNKS // NW
NEXTRA = NCHUNKS - CPT * NW
N_PAD = 10112
STRIPE = N_PAD // NS


def _sc_scatter_partials(edge_features, edge_index, zeros):
    mesh = plsc.VectorSubcoreMesh(core_axis_name="c", subcore_axis_name="s")

    @functools.partial(
        pl.kernel,
        out_type=jax.ShapeDtypeStruct((NC, N_PAD, D_FEAT), jnp.float32),
        mesh=mesh,
        scratch_types=[
            pltpu.VMEM((CPT + 1, CH), jnp.int32),
            pltpu.VMEM((CH, D_FEAT), jnp.float32),
            pltpu.VMEM((CH, D_FEAT), jnp.float32),
            pltpu.VMEM_SHARED((N_PAD, D_FEAT), jnp.float32),
            pltpu.SemaphoreType.DMA,
            pltpu.SemaphoreType.DMA,
            pltpu.SemaphoreType.DMA,
            pltpu.SemaphoreType.DMA,
            pltpu.SemaphoreType.DMA,
        ],
    )
    def k(ef_hbm, ei_hbm, zeros_hbm, out_hbm, idx_v, b0, b1, acc,
          isem, rsem0, rsem1, ssem0, ssem1):
        c = lax.axis_index("c")
        s = lax.axis_index("s")
        wid = s * NC + c
        r0 = s * STRIPE
        cbase = wid * CPT
        xcid = NW * CPT + wid
        bufs = (b0, b1)
        rsems = (rsem0, rsem1)
        ssems = (ssem0, ssem1)

        def idx_slice(cid):
            return ei_hbm.at[0, pl.ds(cid * CH, CH)]

        def edge_slice(cid):
            return ef_hbm.at[pl.ds(cid * CH, CH)]

        pltpu.async_copy(edge_slice(cbase), b0, rsem0)
        for j in range(CPT):
            pltpu.async_copy(idx_slice(cbase + j), idx_v.at[j], isem)

        @pl.when(wid < NEXTRA)
        def _():
            pltpu.async_copy(idx_slice(xcid), idx_v.at[CPT], isem)

        pltpu.sync_copy(zeros_hbm, acc.at[pl.ds(r0, STRIPE)])

        for j in range(CPT):
            pltpu.make_async_copy(idx_slice(cbase + j), idx_v.at[j], isem).wait()

        @pl.when(wid < NEXTRA)
        def _():
            pltpu.make_async_copy(idx_slice(xcid), idx_v.at[CPT], isem).wait()

        plsc.subcore_barrier()

        def scatter_desc(j, buf, ssem):
            return pltpu.make_async_copy(buf, acc.at[idx_v.at[j]], ssem)

        def body(p, carry):
            for h in range(2):
                j = 2 * p + h
                h1 = 1 - h
                buf = bufs[h]
                pltpu.make_async_copy(edge_slice(cbase + j), buf, rsems[h]).wait()
                pltpu.async_copy(buf, acc.at[idx_v.at[j]], ssems[h], add=True)
                @pl.when(j >= 1)
                def _():
                    scatter_desc(j - 1, bufs[h1], ssems[h1]).wait()
                @pl.when(j + 1 < CPT)
                def _():
                    pltpu.async_copy(edge_slice(cbase + j + 1), bufs[h1], rsems[h1])
                @pl.when(jnp.logical_and(j + 1 == CPT, wid < NEXTRA))
                def _():
                    pltpu.async_copy(edge_slice(xcid), bufs[h1], rsems[h1])
            return carry

        lax.fori_loop(0, CPT // 2, body, 0)

        scatter_desc(CPT - 1, b1, ssem1).wait()

        @pl.when(wid < NEXTRA)
        def _():
            pltpu.make_async_copy(edge_slice(xcid), b0, rsem0).wait()
            pltpu.sync_copy(b0, acc.at[idx_v.at[CPT]], add=True)

        plsc.subcore_barrier()
        pltpu.sync_copy(acc.at[pl.ds(r0, STRIPE)],
                        out_hbm.at[c, pl.ds(r0, STRIPE)])

    return k(edge_features, edge_index, zeros)


def _tc_combine(partials):
    def body(p_ref, o_ref):
        o_ref[...] = (p_ref[0] + p_ref[1]) * SCALE

    return pl.pallas_call(
        body,
        out_shape=jax.ShapeDtypeStruct((N_NODES, D_FEAT), jnp.float32),
        grid=(2,),
        in_specs=[pl.BlockSpec((2, 5000, D_FEAT), lambda i: (0, i, 0))],
        out_specs=pl.BlockSpec((5000, D_FEAT), lambda i: (i, 0)),
    )(partials)


def kernel(edge_features, edge_index, pos):
    zeros = jnp.zeros((STRIPE, D_FEAT), jnp.float32)
    partials = _sc_scatter_partials(edge_features, edge_index, zeros)
    return _tc_combine(partials)


# speedup vs baseline: 1.1473x; 1.1473x over previous
import functools

import jax
import jax.numpy as jnp
from jax import lax
from jax.experimental import pallas as pl
from jax.experimental.pallas import tpu as pltpu
from jax.experimental.pallas import tpu_sc as plsc

N_NODES = 10000
N_EDGES = 320000
D_FEAT = 128
SCALE = float(32.0 ** -0.5)

NC = 2
NS = 16
NW = NC * NS
CH = 128
NCHUNKS = N_EDGES // CH
CPT = NCHUNKS // NW
NEXTRA = NCHUNKS - CPT * NW
N_PAD = 10112
STRIPE = N_PAD // NS


def _sc_scatter_partials(edge_features, edge_index, zeros):
    mesh = plsc.VectorSubcoreMesh(core_axis_name="c", subcore_axis_name="s")

    @functools.partial(
        pl.kernel,
        out_type=jax.ShapeDtypeStruct((NC, N_PAD, D_FEAT), jnp.float32),
        mesh=mesh,
        scratch_types=[
            pltpu.VMEM((CPT + 1, CH), jnp.int32),
            pltpu.VMEM((CH, D_FEAT), jnp.float32),
            pltpu.VMEM((CH, D_FEAT), jnp.float32),
            pltpu.VMEM_SHARED((N_PAD, D_FEAT), jnp.float32),
            pltpu.SemaphoreType.DMA,
            pltpu.SemaphoreType.DMA,
            pltpu.SemaphoreType.DMA,
        ],
    )
    def k(ef_hbm, ei_hbm, zeros_hbm, out_hbm, idx_v, b0, b1, acc,
          isem, rsem0, rsem1):
        c = lax.axis_index("c")
        s = lax.axis_index("s")
        wid = s * NC + c
        r0 = s * STRIPE
        cbase = wid * CPT
        xcid = NW * CPT + wid
        bufs = (b0, b1)
        rsems = (rsem0, rsem1)

        def idx_slice(cid):
            return ei_hbm.at[0, pl.ds(cid * CH, CH)]

        def edge_slice(cid):
            return ef_hbm.at[pl.ds(cid * CH, CH)]

        for h in range(2):
            pltpu.async_copy(edge_slice(cbase + h), bufs[h], rsems[h])
        for j in range(CPT):
            pltpu.async_copy(idx_slice(cbase + j), idx_v.at[j], isem)

        @pl.when(wid < NEXTRA)
        def _():
            pltpu.async_copy(idx_slice(xcid), idx_v.at[CPT], isem)

        pltpu.sync_copy(zeros_hbm, acc.at[pl.ds(r0, STRIPE)])

        for j in range(CPT):
            pltpu.make_async_copy(idx_slice(cbase + j), idx_v.at[j], isem).wait()

        @pl.when(wid < NEXTRA)
        def _():
            pltpu.make_async_copy(idx_slice(xcid), idx_v.at[CPT], isem).wait()

        plsc.subcore_barrier()

        def body(p, carry):
            for h in range(2):
                j = 2 * p + h
                cid = cbase + j
                buf, rsem = bufs[h], rsems[h]
                pltpu.make_async_copy(edge_slice(cid), buf, rsem).wait()
                pltpu.sync_copy(buf, acc.at[idx_v.at[j]], add=True)
                @pl.when(j + 2 < CPT)
                def _():
                    pltpu.async_copy(edge_slice(cid + 2), buf, rsem)
                @pl.when(jnp.logical_and(j + 2 == CPT, wid < NEXTRA))
                def _():
                    pltpu.async_copy(edge_slice(xcid), buf, rsem)
            return carry

        lax.fori_loop(0, CPT // 2, body, 0)

        @pl.when(wid < NEXTRA)
        def _():
            pltpu.make_async_copy(edge_slice(xcid), b0, rsem0).wait()
            pltpu.sync_copy(b0, acc.at[idx_v.at[CPT]], add=True)

        plsc.subcore_barrier()
        pltpu.sync_copy(acc.at[pl.ds(r0, STRIPE)],
                        out_hbm.at[c, pl.ds(r0, STRIPE)])

    return k(edge_features, edge_index, zeros)


def _tc_combine(partials):
    def body(p_ref, o_ref):
        o_ref[...] = (p_ref[0] + p_ref[1]) * SCALE

    return pl.pallas_call(
        body,
        out_shape=jax.ShapeDtypeStruct((N_NODES, D_FEAT), jnp.float32),
        grid=(2,),
        in_specs=[pl.BlockSpec((2, 5000, D_FEAT), lambda i: (0, i, 0))],
        out_specs=pl.BlockSpec((5000, D_FEAT), lambda i: (i, 0)),
    )(partials)


def kernel(edge_features, edge_index, pos):
    zeros = jnp.zeros((STRIPE, D_FEAT), jnp.float32)
    partials = _sc_scatter_partials(edge_features, edge_index, zeros)
    return _tc_combine(partials)


# speedup vs baseline: 1.1652x; 1.0156x over previous
import functools

import jax
import jax.numpy as jnp
from jax import lax
from jax.experimental import pallas as pl
from jax.experimental.pallas import tpu as pltpu
from jax.experimental.pallas import tpu_sc as plsc

N_NODES = 10000
N_EDGES = 320000
D_FEAT = 128
SCALE = float(32.0 ** -0.5)

NC = 2
NS = 16
NW = NC * NS
CH = 128
NCHUNKS = N_EDGES // CH
CPT = NCHUNKS // NW
NEXTRA = NCHUNKS - CPT * NW
N_PAD = 10112
STRIPE = N_PAD // NS


def _sc_scatter_partials(edge_features, edge_index, zeros):
    mesh = plsc.VectorSubcoreMesh(core_axis_name="c", subcore_axis_name="s")

    @functools.partial(
        pl.kernel,
        out_type=jax.ShapeDtypeStruct((NC, N_PAD, D_FEAT), jnp.float32),
        mesh=mesh,
        scratch_types=[
            pltpu.VMEM((CH,), jnp.int32),
            pltpu.VMEM((CH,), jnp.int32),
            pltpu.VMEM((CH, D_FEAT), jnp.float32),
            pltpu.VMEM((CH, D_FEAT), jnp.float32),
            pltpu.VMEM_SHARED((N_PAD, D_FEAT), jnp.float32),
            pltpu.SemaphoreType.DMA,
            pltpu.SemaphoreType.DMA,
            pltpu.SemaphoreType.DMA,
            pltpu.SemaphoreType.DMA,
        ],
    )
    def k(ef_hbm, ei_hbm, zeros_hbm, out_hbm, ir0, ir1, b0, b1, acc,
          isem0, isem1, rsem0, rsem1):
        c = lax.axis_index("c")
        s = lax.axis_index("s")
        wid = s * NC + c
        r0 = s * STRIPE
        cbase = wid * CPT
        xcid = NW * CPT + wid
        irs, bufs = (ir0, ir1), (b0, b1)
        isems, rsems = (isem0, isem1), (rsem0, rsem1)

        def idx_slice(cid):
            return ei_hbm.at[0, pl.ds(cid * CH, CH)]

        def edge_slice(cid):
            return ef_hbm.at[pl.ds(cid * CH, CH)]

        for h in range(2):
            pltpu.async_copy(idx_slice(cbase + h), irs[h], isems[h])
            pltpu.async_copy(edge_slice(cbase + h), bufs[h], rsems[h])
        pltpu.sync_copy(zeros_hbm, acc.at[pl.ds(r0, STRIPE)])
        plsc.subcore_barrier()

        def body(p, carry):
            for h in range(2):
                j = 2 * p + h
                cid = cbase + j
                ir, buf = irs[h], bufs[h]
                pltpu.make_async_copy(idx_slice(cid), ir, isems[h]).wait()
                pltpu.make_async_copy(edge_slice(cid), buf, rsems[h]).wait()
                pltpu.sync_copy(buf, acc.at[ir], add=True)
                @pl.when(j + 2 < CPT)
                def _():
                    pltpu.async_copy(idx_slice(cid + 2), ir, isems[h])
                    pltpu.async_copy(edge_slice(cid + 2), buf, rsems[h])
                @pl.when(jnp.logical_and(j + 2 == CPT, wid < NEXTRA))
                def _():
                    pltpu.async_copy(idx_slice(xcid), ir, isems[h])
                    pltpu.async_copy(edge_slice(xcid), buf, rsems[h])
            return carry

        lax.fori_loop(0, CPT // 2, body, 0)

        @pl.when(wid < NEXTRA)
        def _():
            pltpu.make_async_copy(idx_slice(xcid), ir0, isem0).wait()
            pltpu.make_async_copy(edge_slice(xcid), b0, rsem0).wait()
            pltpu.sync_copy(b0, acc.at[ir0], add=True)

        plsc.subcore_barrier()
        pltpu.sync_copy(acc.at[pl.ds(r0, STRIPE)],
                        out_hbm.at[c, pl.ds(r0, STRIPE)])

    return k(edge_features, edge_index, zeros)


def _tc_combine(partials):
    def body(p_ref, o_ref):
        o_ref[...] = (p_ref[0] + p_ref[1]) * SCALE

    return pl.pallas_call(
        body,
        out_shape=jax.ShapeDtypeStruct((N_NODES, D_FEAT), jnp.float32),
        grid=(2,),
        in_specs=[pl.BlockSpec((2, 5000, D_FEAT), lambda i: (0, i, 0))],
        out_specs=pl.BlockSpec((5000, D_FEAT), lambda i: (i, 0)),
    )(partials)


def kernel(edge_features, edge_index, pos):
    zeros = jnp.zeros((STRIPE, D_FEAT), jnp.float32)
    partials = _sc_scatter_partials(edge_features, edge_index, zeros)
    return _tc_combine(partials)


# speedup vs baseline: 1.2641x; 1.0849x over previous
import functools

import jax
import jax.numpy as jnp
from jax import lax
from jax.experimental import pallas as pl
from jax.experimental.pallas import tpu as pltpu
from jax.experimental.pallas import tpu_sc as plsc

N_NODES = 10000
N_EDGES = 320000
D_FEAT = 128
SCALE = float(32.0 ** -0.5)

NC = 2
NS = 16
NW = NC * NS
CH = 128
NCHUNKS = N_EDGES // CH
CPT = NCHUNKS // NW
NEXTRA = NCHUNKS - CPT * NW
N_PAD = 10112
STRIPE = N_PAD // NS


def _sc_scatter_partials(edge_features, edge_index, zeros):
    mesh = plsc.VectorSubcoreMesh(core_axis_name="c", subcore_axis_name="s")

    @functools.partial(
        pl.kernel,
        out_type=jax.ShapeDtypeStruct((NC, N_PAD, D_FEAT), jnp.float32),
        mesh=mesh,
        scratch_types=[
            pltpu.VMEM((CH,), jnp.int32),
            pltpu.VMEM((CH,), jnp.int32),
            pltpu.VMEM((CH,), jnp.int32),
            pltpu.VMEM((CH, D_FEAT), jnp.float32),
            pltpu.VMEM((CH, D_FEAT), jnp.float32),
            pltpu.VMEM((CH, D_FEAT), jnp.float32),
            pltpu.VMEM_SHARED((N_PAD, D_FEAT), jnp.float32),
            pltpu.SemaphoreType.DMA,
            pltpu.SemaphoreType.DMA,
            pltpu.SemaphoreType.DMA,
            pltpu.SemaphoreType.DMA,
            pltpu.SemaphoreType.DMA,
            pltpu.SemaphoreType.DMA,
        ],
    )
    def k(ef_hbm, ei_hbm, zeros_hbm, out_hbm, ir0, ir1, ir2, b0, b1, b2, acc,
          isem0, isem1, isem2, rsem0, rsem1, rsem2):
        c = lax.axis_index("c")
        s = lax.axis_index("s")
        wid = s * NC + c
        r0 = s * STRIPE
        cbase = wid * CPT
        xcid = NW * CPT + wid
        irs, bufs = (ir0, ir1, ir2), (b0, b1, b2)
        isems, rsems = (isem0, isem1, isem2), (rsem0, rsem1, rsem2)

        def idx_slice(cid):
            return ei_hbm.at[0, pl.ds(cid * CH, CH)]

        def edge_slice(cid):
            return ef_hbm.at[pl.ds(cid * CH, CH)]

        for h in range(3):
            pltpu.async_copy(idx_slice(cbase + h), irs[h], isems[h])
            pltpu.async_copy(edge_slice(cbase + h), bufs[h], rsems[h])
        pltpu.sync_copy(zeros_hbm, acc.at[pl.ds(r0, STRIPE)])
        plsc.subcore_barrier()

        def body(p, carry):
            for h in range(3):
                j = 3 * p + h
                cid = cbase + j
                ir, buf = irs[h], bufs[h]
                pltpu.make_async_copy(idx_slice(cid), ir, isems[h]).wait()
                pltpu.make_async_copy(edge_slice(cid), buf, rsems[h]).wait()
                pltpu.sync_copy(buf, acc.at[ir], add=True)
                @pl.when(j + 3 < CPT)
                def _():
                    pltpu.async_copy(idx_slice(cid + 3), ir, isems[h])
                    pltpu.async_copy(edge_slice(cid + 3), buf, rsems[h])
                @pl.when(jnp.logical_and(j + 3 == CPT, wid < NEXTRA))
                def _():
                    pltpu.async_copy(idx_slice(xcid), ir, isems[h])
                    pltpu.async_copy(edge_slice(xcid), buf, rsems[h])
            return carry

        lax.fori_loop(0, CPT // 3, body, 0)

        @pl.when(wid < NEXTRA)
        def _():
            pltpu.make_async_copy(idx_slice(xcid), ir0, isem0).wait()
            pltpu.make_async_copy(edge_slice(xcid), b0, rsem0).wait()
            pltpu.sync_copy(b0, acc.at[ir0], add=True)

        plsc.subcore_barrier()
        pltpu.sync_copy(acc.at[pl.ds(r0, STRIPE)],
                        out_hbm.at[c, pl.ds(r0, STRIPE)])

    return k(edge_features, edge_index, zeros)


def _tc_combine(partials):
    def body(p_ref, o_ref):
        o_ref[...] = (p_ref[0] + p_ref[1]) * SCALE

    return pl.pallas_call(
        body,
        out_shape=jax.ShapeDtypeStruct((N_NODES, D_FEAT), jnp.float32),
        grid=(2,),
        in_specs=[pl.BlockSpec((2, 5000, D_FEAT), lambda i: (0, i, 0))],
        out_specs=pl.BlockSpec((5000, D_FEAT), lambda i: (i, 0)),
    )(partials)


def kernel(edge_features, edge_index, pos):
    zeros = jnp.zeros((STRIPE, D_FEAT), jnp.float32)
    partials = _sc_scatter_partials(edge_features, edge_index, zeros)
    return _tc_combine(partials)
